# baseline (speedup 1.0000x reference)
import jax
import jax.numpy as jnp
from jax.experimental import pallas as pl
from jax.experimental.pallas import tpu as pltpu


def _x0_body(t_ref, a_ref, c_ref, x_ref, eps_ref, o_ref):
    ti = t_ref[pl.program_id(0)]
    o_ref[...] = a_ref[ti] * x_ref[...] - c_ref[ti] * eps_ref[...]


def kernel(model_preds, x_t, x_0, noise, t,
           sqrt_recip_alphas_cumprod, sqrt_recipm1_alphas_cumprod):
    B = x_t.shape[0]
    n = x_t.size // B
    x2 = x_t.reshape(B, 128, n // 128)
    e2 = model_preds.reshape(B, 128, n // 128)
    grid_spec = pltpu.PrefetchScalarGridSpec(
        num_scalar_prefetch=3,
        grid=(B,),
        in_specs=[
            pl.BlockSpec((1, 128, n // 128), lambda b, *_: (b, 0, 0)),
            pl.BlockSpec((1, 128, n // 128), lambda b, *_: (b, 0, 0)),
        ],
        out_specs=pl.BlockSpec((1, 128, n // 128), lambda b, *_: (b, 0, 0)),
    )
    x0p = pl.pallas_call(
        _x0_body,
        grid_spec=grid_spec,
        out_shape=jax.ShapeDtypeStruct((B, 128, n // 128), x_t.dtype),
    )(t, sqrt_recip_alphas_cumprod, sqrt_recipm1_alphas_cumprod, x2, e2)
    return (model_preds, x0p.reshape(x_t.shape), noise)


# speedup vs baseline: 1.1880x; 1.1880x over previous
import jax
import jax.numpy as jnp
from jax.experimental import pallas as pl
from jax.experimental.pallas import tpu as pltpu

_PB = 4


def _body(t_ref, a_ref, c_ref, x_ref, eps_ref, nz_ref, x0_ref, np_ref, tg_ref):
    g = pl.program_id(0)
    np_ref[...] = eps_ref[...]
    tg_ref[...] = nz_ref[...]
    for j in range(_PB):
        ti = t_ref[g * _PB + j]
        x0_ref[j] = a_ref[ti] * x_ref[j] - c_ref[ti] * eps_ref[j]


def kernel(model_preds, x_t, x_0, noise, t,
           sqrt_recip_alphas_cumprod, sqrt_recipm1_alphas_cumprod):
    B = x_t.shape[0]
    n = x_t.size // B
    shp3 = (B, 128, n // 128)
    x2 = x_t.reshape(shp3)
    e2 = model_preds.reshape(shp3)
    nz2 = noise.reshape(shp3)
    blk = pl.BlockSpec((_PB, 128, n // 128), lambda g, *_: (g, 0, 0))
    grid_spec = pltpu.PrefetchScalarGridSpec(
        num_scalar_prefetch=3,
        grid=(B // _PB,),
        in_specs=[blk, blk, blk],
        out_specs=[blk, blk, blk],
    )
    out = jax.ShapeDtypeStruct(shp3, x_t.dtype)
    x0p, np_, tg = pl.pallas_call(
        _body,
        grid_spec=grid_spec,
        out_shape=[out, out, out],
    )(t, sqrt_recip_alphas_cumprod, sqrt_recipm1_alphas_cumprod, x2, e2, nz2)
    shp = x_t.shape
    return (np_.reshape(shp), x0p.reshape(shp), tg.reshape(shp))


# speedup vs baseline: 1.2865x; 1.0829x over previous
import jax
import jax.numpy as jnp
from jax.experimental import pallas as pl
from jax.experimental.pallas import tpu as pltpu

_PB = 32


def _body(t_ref, a_ref, c_ref, x_ref, eps_ref, nz_ref, x0_ref, np_ref, tg_ref):
    g = pl.program_id(0)
    np_ref[...] = eps_ref[...]
    tg_ref[...] = nz_ref[...]
    for j in range(_PB):
        ti = t_ref[g * _PB + j]
        x0_ref[j] = a_ref[ti] * x_ref[j] - c_ref[ti] * eps_ref[j]


def kernel(model_preds, x_t, x_0, noise, t,
           sqrt_recip_alphas_cumprod, sqrt_recipm1_alphas_cumprod):
    B = x_t.shape[0]
    n = x_t.size // B
    shp3 = (B, 128, n // 128)
    x2 = x_t.reshape(shp3)
    e2 = model_preds.reshape(shp3)
    nz2 = noise.reshape(shp3)
    blk = pl.BlockSpec((_PB, 128, n // 128), lambda g, *_: (g, 0, 0))
    grid_spec = pltpu.PrefetchScalarGridSpec(
        num_scalar_prefetch=3,
        grid=(B // _PB,),
        in_specs=[blk, blk, blk],
        out_specs=[blk, blk, blk],
    )
    out = jax.ShapeDtypeStruct(shp3, x_t.dtype)
    x0p, np_, tg = pl.pallas_call(
        _body,
        grid_spec=grid_spec,
        out_shape=[out, out, out],
    )(t, sqrt_recip_alphas_cumprod, sqrt_recipm1_alphas_cumprod, x2, e2, nz2)
    shp = x_t.shape
    return (np_.reshape(shp), x0p.reshape(shp), tg.reshape(shp))


# speedup vs baseline: 2.4224x; 1.8829x over previous
import jax
import jax.numpy as jnp
from jax.experimental import pallas as pl
from jax.experimental.pallas import tpu as pltpu


def _gather_body(t_ref, a_ref, c_ref, ga_ref, gc_ref):
    for j in range(32):
        ti = t_ref[j]
        ga_ref[j] = a_ref[ti]
        gc_ref[j] = c_ref[ti]


def kernel(model_preds, x_t, x_0, noise, t,
           sqrt_recip_alphas_cumprod, sqrt_recipm1_alphas_cumprod):
    B = x_t.shape[0]
    grid_spec = pltpu.PrefetchScalarGridSpec(
        num_scalar_prefetch=3,
        grid=(1,),
        in_specs=[],
        out_specs=[
            pl.BlockSpec(memory_space=pltpu.SMEM),
            pl.BlockSpec(memory_space=pltpu.SMEM),
        ],
    )
    ga, gc = pl.pallas_call(
        _gather_body,
        grid_spec=grid_spec,
        out_shape=[jax.ShapeDtypeStruct((B,), jnp.float32),
                   jax.ShapeDtypeStruct((B,), jnp.float32)],
    )(t, sqrt_recip_alphas_cumprod, sqrt_recipm1_alphas_cumprod)
    x0 = ga[:, None, None, None] * x_t - gc[:, None, None, None] * model_preds
    return (model_preds, x0, noise)


# speedup vs baseline: 3.2312x; 1.3339x over previous
import jax
import jax.numpy as jnp
from jax.experimental import pallas as pl
from jax.experimental.pallas import tpu as pltpu

_PB = 8


def _body(t_ref, a_ref, c_ref, x_ref, eps_ref, nz_ref, x0_ref, np_ref, tg_ref):
    g = pl.program_id(0)
    np_ref[...] = eps_ref[...]
    tg_ref[...] = nz_ref[...]
    for j in range(_PB):
        ti = t_ref[g * _PB + j]
        x0_ref[j] = a_ref[ti] * x_ref[j] - c_ref[ti] * eps_ref[j]


def kernel(model_preds, x_t, x_0, noise, t,
           sqrt_recip_alphas_cumprod, sqrt_recipm1_alphas_cumprod):
    B, C, H, W = x_t.shape
    blk = pl.BlockSpec((_PB, C, H, W), lambda g, *_: (g, 0, 0, 0))
    grid_spec = pltpu.PrefetchScalarGridSpec(
        num_scalar_prefetch=3,
        grid=(B // _PB,),
        in_specs=[blk, blk, blk],
        out_specs=[blk, blk, blk],
    )
    out = jax.ShapeDtypeStruct(x_t.shape, x_t.dtype)
    x0p, np_, tg = pl.pallas_call(
        _body,
        grid_spec=grid_spec,
        out_shape=[out, out, out],
    )(t, sqrt_recip_alphas_cumprod, sqrt_recipm1_alphas_cumprod,
      x_t, model_preds, noise)
    return (np_, x0p, tg)
